# Initial kernel scaffold; baseline (speedup 1.0000x reference)
import jax
import jax.numpy as jnp
from jax.experimental import pallas as pl


def kernel(x, timestamps, seq_lens, token):
    raise NotImplementedError("write your pallas kernel here")


# speedup vs baseline: 1.2974x; 1.2974x over previous
import jax
import jax.numpy as jnp
from jax.experimental import pallas as pl

B, L, D = 16, 512, 1024
CHUNK = 256
MROWS = 32


def _interleave_body(x_ref, tok_ref, o_ref):
    o_ref[0, :, 0:D] = x_ref[0]
    o_ref[0, :, D:2 * D] = jnp.broadcast_to(tok_ref[...], (CHUNK, D))


def _small_body(ns_ref, ts_ref, seq_ref, mask_ref, ts3_ref, len_ref):
    i = pl.program_id(0)
    ncol = 2 * L // 4
    r = i * MROWS + jax.lax.broadcasted_iota(jnp.int32, (MROWS, ncol), 0)
    j = jax.lax.broadcasted_iota(jnp.int32, (MROWS, ncol), 1)
    n2 = ns_ref[...] * 2
    nm1 = jnp.maximum(n2 - 1, 0)
    re = 1 - (r & 1)

    def _nz(d):
        return ((d | -d) >> 31) & 1

    def mbit(c):
        odd_c = c & 1
        lt = ((c - n2) >> 31) & 1
        m_even = (lt | odd_c) & _nz(c - nm1)
        m_odd = odd_c & _nz(c - r)
        return re * m_even + (1 - re) * m_odd

    packed = (mbit(4 * j) | (mbit(4 * j + 1) << 8)
              | (mbit(4 * j + 2) << 16) | (mbit(4 * j + 3) << 24))
    mask_ref[...] = packed

    @pl.when(i == 0)
    def _():
        ts = ts_ref[...]
        ts3_ref[:, :, 0] = ts
        ts3_ref[:, :, 1] = ts
        len_ref[...] = seq_ref[...] * 2


def kernel(x, timestamps, seq_lens, token):
    mk = jax.random.key(42)
    ka, kb = jax.random.split(mk)
    n_summarize = jnp.round(
        jax.random.uniform(ka, (L,)) * jnp.arange(L, dtype=jnp.float32)
    ).astype(jnp.int32)
    gate = jax.random.uniform(kb, ())
    n_summarize = jnp.where(gate > 0.5, jnp.zeros_like(n_summarize), n_summarize)
    nsrep = jnp.repeat(n_summarize, 2).reshape(2 * L, 1)

    big = pl.pallas_call(
        _interleave_body,
        grid=(B, L // CHUNK),
        in_specs=[
            pl.BlockSpec((1, CHUNK, D), lambda b, l: (b, l, 0)),
            pl.BlockSpec((1, D), lambda b, l: (0, 0)),
        ],
        out_specs=pl.BlockSpec((1, CHUNK, 2 * D), lambda b, l: (b, l, 0)),
        out_shape=jax.ShapeDtypeStruct((B, L, 2 * D), jnp.float32),
    )(x, token.reshape(1, D))
    new_x = big.reshape(B, 2 * L, D)

    mask, ts3, len2 = pl.pallas_call(
        _small_body,
        grid=(2 * L // MROWS,),
        in_specs=[
            pl.BlockSpec((MROWS, 1), lambda i: (i, 0)),
            pl.BlockSpec((B, L), lambda i: (0, 0)),
            pl.BlockSpec((1, B), lambda i: (0, 0)),
        ],
        out_specs=[
            pl.BlockSpec((MROWS, 2 * L // 4), lambda i: (i, 0)),
            pl.BlockSpec((B, L, 2), lambda i: (0, 0, 0)),
            pl.BlockSpec((1, B), lambda i: (0, 0)),
        ],
        out_shape=[
            jax.ShapeDtypeStruct((2 * L, 2 * L // 4), jnp.int32),
            jax.ShapeDtypeStruct((B, L, 2), jnp.float32),
            jax.ShapeDtypeStruct((1, B), jnp.int32),
        ],
    )(nsrep, timestamps, seq_lens.reshape(1, B))

    new_timestamps = ts3.reshape(B, 2 * L)
    new_lengths = len2.reshape(B)
    mask_bytes = jax.lax.bitcast_convert_type(mask, jnp.int8)
    attention_mask = mask_bytes.reshape(2 * L, 2 * L).astype(jnp.bool_)
    return (new_x, new_timestamps, new_lengths, attention_mask)
